# Initial kernel scaffold; baseline (speedup 1.0000x reference)
import jax
import jax.numpy as jnp
from jax.experimental import pallas as pl


def kernel(sub_x, sub_edge_index, sub_batch, sub_index, global_x, global_edge_index, global_batch, W_sub, b_sub, W_glob, b_glob, W_fc, b_fc):
    raise NotImplementedError("write your pallas kernel here")


# speedup vs baseline: 12.5129x; 12.5129x over previous
import functools

import jax
import jax.numpy as jnp
from jax import lax
from jax.experimental import pallas as pl
from jax.experimental.pallas import tpu as pltpu
from jax.experimental.pallas import tpu_sc as plsc

N_NODE = 10000
D_FEAT = 128
N_ACC = 10112
ROWS_PER_TILE = N_ACC // 16
PAD_IDX = 10000
N_BATCH = 16
CHUNK = 128
N_WORKERS = 32


def _pad_edges(edge_index, e_pad):
    e = edge_index.shape[1]
    pad = e_pad - e
    padv = jnp.full((pad,), PAD_IDX, jnp.int32)
    src = jnp.concatenate([edge_index[0], padv])
    dst = jnp.concatenate([edge_index[1], padv])
    return src, dst


def _sc_mesh():
    return plsc.VectorSubcoreMesh(core_axis_name="c", subcore_axis_name="s")


def _deg_body(cpt_sub, cpt_glob, dst_sub, dst_glob, out_hbm,
              idx_v, ones_v, zbuf_v, deg_sub_sh, deg_glob_sh):
    c = lax.axis_index("c")
    s = lax.axis_index("s")

    def fill(i, _):
        ones_v[pl.ds(i * 16, 16)] = jnp.ones((16,), jnp.float32)
        return 0
    lax.fori_loop(0, CHUNK // 16, fill, 0)

    def zfill(i, _):
        zbuf_v[pl.ds(i * 16, 16)] = jnp.zeros((16,), jnp.float32)
        return 0
    lax.fori_loop(0, N_ACC // 16, zfill, 0)

    @pl.when(s == 0)
    def _():
        pltpu.sync_copy(zbuf_v, deg_sub_sh)
        pltpu.sync_copy(zbuf_v, deg_glob_sh)
    plsc.subcore_barrier()

    wid = c * 16 + s

    def scatter_graph(dst_hbm, deg_sh, cpt):
        def body(k, _):
            off = (wid * cpt + k) * CHUNK
            pltpu.sync_copy(dst_hbm.at[pl.ds(off, CHUNK)], idx_v)
            pltpu.sync_copy(ones_v, deg_sh.at[idx_v], add=True)
            return 0
        lax.fori_loop(0, cpt, body, 0)

    scatter_graph(dst_sub, deg_sub_sh, cpt_sub)
    scatter_graph(dst_glob, deg_glob_sh, cpt_glob)
    plsc.subcore_barrier()

    @pl.when(jnp.logical_and(s == 0, c == 0))
    def _():
        pltpu.sync_copy(deg_sub_sh, out_hbm.at[0, 0])
        pltpu.sync_copy(deg_glob_sh, out_hbm.at[1, 0])

    @pl.when(jnp.logical_and(s == 0, c == 1))
    def _():
        pltpu.sync_copy(deg_sub_sh, out_hbm.at[0, 1])
        pltpu.sync_copy(deg_glob_sh, out_hbm.at[1, 1])


def _make_deg_kernel(e_pad_sub, e_pad_glob):
    cpt_sub = e_pad_sub // (N_WORKERS * CHUNK)
    cpt_glob = e_pad_glob // (N_WORKERS * CHUNK)
    return pl.kernel(
        functools.partial(_deg_body, cpt_sub, cpt_glob),
        out_type=jax.ShapeDtypeStruct((2, 2, N_ACC), jnp.float32),
        mesh=_sc_mesh(),
        scratch_types=[
            pltpu.VMEM((CHUNK,), jnp.int32),
            pltpu.VMEM((CHUNK,), jnp.float32),
            pltpu.VMEM((N_ACC,), jnp.float32),
            pltpu.VMEM_SHARED((N_ACC,), jnp.float32),
            pltpu.VMEM_SHARED((N_ACC,), jnp.float32),
        ],
    )


def _rows_body(cpt, h_hbm, src_hbm, dst_hbm, zeros_hbm, out_hbm,
               idx_s, idx_d, rows_v, zb_v, acc_sh, sem):
    c = lax.axis_index("c")
    s = lax.axis_index("s")

    pltpu.sync_copy(zeros_hbm, zb_v)
    base = s * ROWS_PER_TILE
    for j in range(4):
        pltpu.sync_copy(zb_v, acc_sh.at[pl.ds(base + j * CHUNK, CHUNK)])
    pltpu.sync_copy(zb_v.at[pl.ds(0, ROWS_PER_TILE - 4 * CHUNK)],
                    acc_sh.at[pl.ds(base + 4 * CHUNK, ROWS_PER_TILE - 4 * CHUNK)])
    plsc.subcore_barrier()

    wid = c * 16 + s

    def body(k, _):
        off = (wid * cpt + k) * CHUNK
        pltpu.sync_copy(src_hbm.at[pl.ds(off, CHUNK)], idx_s)
        pltpu.sync_copy(dst_hbm.at[pl.ds(off, CHUNK)], idx_d)
        pltpu.async_copy(h_hbm.at[idx_s], rows_v, sem).wait()
        pltpu.sync_copy(rows_v, acc_sh.at[idx_d], add=True)
        return 0
    lax.fori_loop(0, cpt, body, 0)
    plsc.subcore_barrier()

    sizes = [CHUNK] * 4 + [ROWS_PER_TILE - 4 * CHUNK]

    @pl.when(c == 0)
    def _():
        o = 0
        for sz in sizes:
            pltpu.sync_copy(acc_sh.at[pl.ds(base + o, sz)],
                            out_hbm.at[0, pl.ds(base + o, sz)])
            o += sz

    @pl.when(c == 1)
    def _():
        o = 0
        for sz in sizes:
            pltpu.sync_copy(acc_sh.at[pl.ds(base + o, sz)],
                            out_hbm.at[1, pl.ds(base + o, sz)])
            o += sz


def _make_rows_kernel(e_pad):
    cpt = e_pad // (N_WORKERS * CHUNK)
    return pl.kernel(
        functools.partial(_rows_body, cpt),
        out_type=jax.ShapeDtypeStruct((2, N_ACC, D_FEAT), jnp.float32),
        mesh=_sc_mesh(),
        scratch_types=[
            pltpu.VMEM((CHUNK,), jnp.int32),
            pltpu.VMEM((CHUNK,), jnp.int32),
            pltpu.VMEM((CHUNK, D_FEAT), jnp.float32),
            pltpu.VMEM((CHUNK, D_FEAT), jnp.float32),
            pltpu.VMEM_SHARED((N_ACC, D_FEAT), jnp.float32),
            pltpu.SemaphoreType.DMA,
        ],
    )


def _dinv(degp_ref, g):
    deg = degp_ref[g, 0, 0:N_NODE, :] + degp_ref[g, 1, 0:N_NODE, :] + 1.0
    return lax.rsqrt(jnp.maximum(deg, 1e-12))


def _mm_sub_body(x_ref, w_ref, degp_ref, o_ref):
    h = jnp.dot(x_ref[:], w_ref[:], preferred_element_type=jnp.float32)
    o_ref[0:N_NODE, :] = h * _dinv(degp_ref, 0)
    o_ref[N_NODE:N_NODE + 8, :] = jnp.zeros((8, D_FEAT), jnp.float32)


def _fin_sub_body(hsub_ref, acc_ref, degp_ref, b_ref, batch_ref, o_ref):
    dinv = _dinv(degp_ref, 0)
    pre = (hsub_ref[0:N_NODE, :] + acc_ref[0, 0:N_NODE, :]
           + acc_ref[1, 0:N_NODE, :]) * dinv + b_ref[:]
    hs = jnp.maximum(pre, 0.0)
    onehot = (batch_ref[:] == lax.broadcasted_iota(
        jnp.int32, (N_NODE, N_BATCH), 1)).astype(jnp.float32)
    psum = lax.dot_general(onehot, hs, (((0,), (0,)), ((), ())),
                           preferred_element_type=jnp.float32)
    cnt = lax.dot_general(onehot, jnp.ones((N_NODE, 1), jnp.float32),
                          (((0,), (0,)), ((), ())),
                          preferred_element_type=jnp.float32)
    o_ref[:] = psum / jnp.maximum(cnt, 1.0)


def _mm_glob_body(x_ref, w_ref, degp_ref, pooled_ref, sidx_ref, o_ref):
    h = jnp.dot(x_ref[:], w_ref[:], preferred_element_type=jnp.float32)
    o_ref[0:N_NODE, :] = h
    for j in range(N_BATCH):
        r = (sidx_ref[j] - 1) % N_NODE
        xr = x_ref[pl.ds(r, 1), :] + pooled_ref[pl.ds(j, 1), :]
        o_ref[pl.ds(r, 1), :] = jnp.dot(xr, w_ref[:],
                                        preferred_element_type=jnp.float32)
    o_ref[0:N_NODE, :] = o_ref[0:N_NODE, :] * _dinv(degp_ref, 1)
    o_ref[N_NODE:N_NODE + 8, :] = jnp.zeros((8, D_FEAT), jnp.float32)


def _fin_glob_body(hg_ref, acc_ref, degp_ref, b_ref, wfc_ref, bfc_ref, o_ref):
    dinv = _dinv(degp_ref, 1)
    pre = (hg_ref[0:N_NODE, :] + acc_ref[0, 0:N_NODE, :]
           + acc_ref[1, 0:N_NODE, :]) * dinv + b_ref[:]
    hg = jnp.maximum(pre, 0.0)
    emb = jnp.sum(hg, axis=0, keepdims=True) / jnp.float32(N_NODE)
    o_ref[:] = jnp.dot(emb, wfc_ref[:],
                       preferred_element_type=jnp.float32) + bfc_ref[:]


def _tc_call(body, out_shape, n_in, smem_args=()):
    in_specs = [pl.BlockSpec(memory_space=pltpu.VMEM) for _ in range(n_in)]
    for i in smem_args:
        in_specs[i] = pl.BlockSpec(memory_space=pltpu.SMEM)
    return pl.pallas_call(body, out_shape=out_shape, in_specs=in_specs,
                          out_specs=pl.BlockSpec(memory_space=pltpu.VMEM))


def _round_up(x, m):
    return ((x + m - 1) // m) * m


@jax.jit
def kernel(sub_x, sub_edge_index, sub_batch, sub_index, global_x,
           global_edge_index, global_batch, W_sub, b_sub, W_glob, b_glob,
           W_fc, b_fc):
    e_sub = sub_edge_index.shape[1]
    e_glob = global_edge_index.shape[1]
    ep_sub = _round_up(e_sub, N_WORKERS * CHUNK)
    ep_glob = _round_up(e_glob, N_WORKERS * CHUNK)

    src_s, dst_s = _pad_edges(sub_edge_index, ep_sub)
    src_g, dst_g = _pad_edges(global_edge_index, ep_glob)
    zeros_blk = jnp.zeros((CHUNK, D_FEAT), jnp.float32)

    degp = _make_deg_kernel(ep_sub, ep_glob)(dst_s, dst_g)
    degp = degp.reshape(2, 2, N_ACC, 1)

    hsub = _tc_call(_mm_sub_body,
                    jax.ShapeDtypeStruct((N_NODE + 8, D_FEAT), jnp.float32),
                    3)(sub_x, W_sub, degp)

    acc_s = _make_rows_kernel(ep_sub)(hsub, src_s, dst_s, zeros_blk)

    pooled = _tc_call(_fin_sub_body,
                      jax.ShapeDtypeStruct((N_BATCH, D_FEAT), jnp.float32),
                      5)(hsub, acc_s, degp, b_sub.reshape(1, D_FEAT),
                         sub_batch.reshape(N_NODE, 1))

    hglob = _tc_call(_mm_glob_body,
                     jax.ShapeDtypeStruct((N_NODE + 8, D_FEAT), jnp.float32),
                     5, smem_args=(4,))(global_x, W_glob, degp, pooled,
                                        sub_index)

    acc_g = _make_rows_kernel(ep_glob)(hglob, src_g, dst_g, zeros_blk)

    out = _tc_call(_fin_glob_body,
                   jax.ShapeDtypeStruct((1, D_FEAT), jnp.float32),
                   6)(hglob, acc_g, degp, b_glob.reshape(1, D_FEAT),
                      W_fc, b_fc.reshape(1, D_FEAT))
    return out
